# baseline (speedup 1.0000x reference)
import functools
import math

import jax
import jax.numpy as jnp
import ml_dtypes
import numpy as np
from jax import lax
from jax.experimental import pallas as pl
from jax.experimental.pallas import tpu as pltpu
from jax.experimental.pallas import tpu_sc as plsc

VOCAB = 1000000
SEQ_LEN = 2048
D_EMBED = 128
BATCH = 4
SCALE = math.sqrt(float(D_EMBED))

NUM_CORES = 2
NUM_SUBCORES = 16
NW = NUM_CORES * NUM_SUBCORES
B_TOTAL = BATCH * SEQ_LEN
B_PER_W = B_TOTAL // NW
W_PER_BATCH = SEQ_LEN // B_PER_W
N_CHUNKS = 4
C_ROWS = B_PER_W // N_CHUNKS
LANES = 16


def _positional_table() -> np.ndarray:
    pos = np.arange(SEQ_LEN)[:, None].astype(np.float32)
    i = np.arange(D_EMBED)[None, :].astype(np.float32)
    angle_rates = 1.0 / np.power(
        10000.0, (2.0 * np.floor(i / 2.0)) / float(D_EMBED))
    angles = pos * angle_rates
    pe = np.zeros((SEQ_LEN, D_EMBED), dtype=np.float32)
    pe[:, 0::2] = np.sin(angles[:, 0::2])
    pe[:, 1::2] = np.cos(angles[:, 1::2])
    perm = pe.reshape(SEQ_LEN, D_EMBED // 32, 2, LANES)
    perm = perm.transpose(0, 1, 3, 2).reshape(SEQ_LEN * D_EMBED)
    bf = perm.astype(ml_dtypes.bfloat16)
    u16 = bf.view(np.uint16).reshape(-1, 2).astype(np.uint32)
    return u16[:, 0] | (u16[:, 1] << np.uint32(16))


_PE_BF16 = _positional_table()


def _make_sc_kernel():
    mesh = plsc.VectorSubcoreMesh(
        core_axis_name="c", subcore_axis_name="s")

    @functools.partial(
        pl.kernel,
        mesh=mesh,
        out_type=jax.ShapeDtypeStruct((BATCH, SEQ_LEN, D_EMBED), jnp.float32),
        scratch_types=[
            pltpu.VMEM((B_PER_W,), jnp.int32),
            pltpu.VMEM((N_CHUNKS, C_ROWS, D_EMBED), jnp.float32),
            pltpu.VMEM((B_PER_W * D_EMBED // 2,), jnp.uint32),
            pltpu.SemaphoreType.DMA,
            pltpu.SemaphoreType.DMA,
            pltpu.SemaphoreType.DMA,
            pltpu.SemaphoreType.DMA,
            pltpu.SemaphoreType.DMA,
            pltpu.SemaphoreType.DMA,
        ],
    )
    def emb_kernel(table_hbm, idx_hbm, pe_hbm, out_hbm,
                   idx_v, rows_v, pe_v, g0, g1, g2, g3, pe_sem, w_sem):
        wid = lax.axis_index("s") * NUM_CORES + lax.axis_index("c")
        b = wid // W_PER_BATCH
        col0 = lax.rem(wid, W_PER_BATCH) * B_PER_W
        pltpu.sync_copy(idx_hbm.at[b, pl.ds(col0, B_PER_W)], idx_v)
        gsems = [g0, g1, g2, g3]
        gathers = [
            pltpu.async_copy(
                table_hbm.at[idx_v.at[pl.ds(c * C_ROWS, C_ROWS)]],
                rows_v.at[c],
                gsems[c],
            )
            for c in range(N_CHUNKS)
        ]
        pe_off = pl.multiple_of(col0 * (D_EMBED // 2), 8)
        pe_cp = pltpu.async_copy(
            pe_hbm.at[pl.ds(pe_off, B_PER_W * D_EMBED // 2)],
            pe_v, pe_sem)
        pe_cp.wait()
        writes = []
        for c in range(N_CHUNKS):
            gathers[c].wait()

            def row_body(r, carry, c=c):
                pe_base = (c * C_ROWS + r) * (D_EMBED // 2)
                for j in range(D_EMBED // (2 * LANES)):
                    u = pe_v[pl.ds(pe_base + j * LANES, LANES)]
                    pa = lax.bitcast_convert_type(u << 16, jnp.float32)
                    pb = lax.bitcast_convert_type(
                        u & jnp.uint32(0xFFFF0000), jnp.float32)
                    sl_a = pl.ds(j * 2 * LANES, LANES)
                    sl_b = pl.ds(j * 2 * LANES + LANES, LANES)
                    rows_v[c, r, sl_a] = rows_v[c, r, sl_a] * SCALE + pa
                    rows_v[c, r, sl_b] = rows_v[c, r, sl_b] * SCALE + pb
                return carry

            lax.fori_loop(0, C_ROWS, row_body, 0)
            writes.append(pltpu.async_copy(
                rows_v.at[c],
                out_hbm.at[b, pl.ds(col0 + c * C_ROWS, C_ROWS)],
                w_sem,
            ))
        for w in writes:
            w.wait()

    return emb_kernel


_EMB_KERNEL = _make_sc_kernel()


def kernel(input, token_table):
    pe = jnp.asarray(_PE_BF16)
    return _EMB_KERNEL(token_table, input, pe)


# speedup vs baseline: 1.0711x; 1.0711x over previous
import functools
import math

import jax
import jax.numpy as jnp
import numpy as np
from jax import lax
from jax.experimental import pallas as pl
from jax.experimental.pallas import tpu as pltpu
from jax.experimental.pallas import tpu_sc as plsc

VOCAB = 1000000
SEQ_LEN = 2048
D_EMBED = 128
BATCH = 4
SCALE = math.sqrt(float(D_EMBED))

NUM_CORES = 2
NUM_SUBCORES = 16
NW = NUM_CORES * NUM_SUBCORES
B_TOTAL = BATCH * SEQ_LEN
B_PER_W = B_TOTAL // NW
W_PER_BATCH = SEQ_LEN // B_PER_W
N_CHUNKS = 4
C_ROWS = B_PER_W // N_CHUNKS
LANES = 16


def _positional_table() -> np.ndarray:
    pos = np.arange(SEQ_LEN)[:, None].astype(np.float32)
    i = np.arange(D_EMBED)[None, :].astype(np.float32)
    angle_rates = 1.0 / np.power(
        10000.0, (2.0 * np.floor(i / 2.0)) / float(D_EMBED))
    angles = pos * angle_rates
    pe = np.zeros((SEQ_LEN, D_EMBED), dtype=np.float32)
    pe[:, 0::2] = np.sin(angles[:, 0::2])
    pe[:, 1::2] = np.cos(angles[:, 1::2])
    return pe.reshape(SEQ_LEN * D_EMBED)


_PE_BF16 = _positional_table()


def _make_sc_kernel():
    mesh = plsc.VectorSubcoreMesh(
        core_axis_name="c", subcore_axis_name="s")

    @functools.partial(
        pl.kernel,
        mesh=mesh,
        out_type=jax.ShapeDtypeStruct((BATCH, SEQ_LEN, D_EMBED), jnp.float32),
        scratch_types=[
            pltpu.VMEM((B_PER_W,), jnp.int32),
            pltpu.VMEM((N_CHUNKS, C_ROWS, D_EMBED), jnp.float32),
            pltpu.VMEM((B_PER_W * D_EMBED,), jnp.float32),
            pltpu.SemaphoreType.DMA,
            pltpu.SemaphoreType.DMA,
            pltpu.SemaphoreType.DMA,
            pltpu.SemaphoreType.DMA,
            pltpu.SemaphoreType.DMA,
            pltpu.SemaphoreType.DMA,
        ],
    )
    def emb_kernel(table_hbm, idx_hbm, pe_hbm, out_hbm,
                   idx_v, rows_v, pe_v, g0, g1, g2, g3, pe_sem, w_sem):
        wid = lax.axis_index("s") * NUM_CORES + lax.axis_index("c")
        b = wid // W_PER_BATCH
        col0 = lax.rem(wid, W_PER_BATCH) * B_PER_W
        pltpu.sync_copy(idx_hbm.at[b, pl.ds(col0, B_PER_W)], idx_v)
        gsems = [g0, g1, g2, g3]
        gathers = [
            pltpu.async_copy(
                table_hbm.at[idx_v.at[pl.ds(c * C_ROWS, C_ROWS)]],
                rows_v.at[c],
                gsems[c],
            )
            for c in range(N_CHUNKS)
        ]
        pe_off = pl.multiple_of(col0 * D_EMBED, 8)
        pe_cp = pltpu.async_copy(
            pe_hbm.at[pl.ds(pe_off, B_PER_W * D_EMBED)],
            pe_v, pe_sem)
        pe_cp.wait()
        writes = []
        for c in range(N_CHUNKS):
            gathers[c].wait()

            @plsc.parallel_loop(0, C_ROWS, unroll=4)
            def row_body(r, c=c):
                pe_base = (c * C_ROWS + r) * D_EMBED
                for j in range(D_EMBED // LANES):
                    sl = pl.ds(j * LANES, LANES)
                    rows_v[c, r, sl] = (rows_v[c, r, sl] * SCALE
                                        + pe_v[pl.ds(pe_base + j * LANES,
                                                     LANES)])
            writes.append(pltpu.async_copy(
                rows_v.at[c],
                out_hbm.at[b, pl.ds(col0 + c * C_ROWS, C_ROWS)],
                w_sem,
            ))
        for w in writes:
            w.wait()

    return emb_kernel


_EMB_KERNEL = _make_sc_kernel()


def kernel(input, token_table):
    pe = jnp.asarray(_PE_BF16)
    return _EMB_KERNEL(token_table, input, pe)


# speedup vs baseline: 1.0971x; 1.0243x over previous
import functools
import math

import jax
import jax.numpy as jnp
import numpy as np
from jax import lax
from jax.experimental import pallas as pl
from jax.experimental.pallas import tpu as pltpu
from jax.experimental.pallas import tpu_sc as plsc

VOCAB = 1000000
SEQ_LEN = 2048
D_EMBED = 128
BATCH = 4
SCALE = math.sqrt(float(D_EMBED))

NUM_CORES = 2
NUM_SUBCORES = 16
NW = NUM_CORES * NUM_SUBCORES
B_TOTAL = BATCH * SEQ_LEN
B_PER_W = B_TOTAL // NW
W_PER_BATCH = SEQ_LEN // B_PER_W
N_CHUNKS = 4
C_ROWS = B_PER_W // N_CHUNKS
LANES = 16


def _positional_table() -> np.ndarray:
    pos = np.arange(SEQ_LEN)[:, None].astype(np.float32)
    i = np.arange(D_EMBED)[None, :].astype(np.float32)
    angle_rates = 1.0 / np.power(
        10000.0, (2.0 * np.floor(i / 2.0)) / float(D_EMBED))
    angles = pos * angle_rates
    pe = np.zeros((SEQ_LEN, D_EMBED), dtype=np.float32)
    pe[:, 0::2] = np.sin(angles[:, 0::2])
    pe[:, 1::2] = np.cos(angles[:, 1::2])
    return pe.reshape(SEQ_LEN * D_EMBED)


_PE_BF16 = _positional_table()


def _make_sc_kernel():
    mesh = plsc.VectorSubcoreMesh(
        core_axis_name="c", subcore_axis_name="s")

    @functools.partial(
        pl.kernel,
        mesh=mesh,
        out_type=jax.ShapeDtypeStruct((BATCH, SEQ_LEN, D_EMBED), jnp.float32),
        scratch_types=[
            pltpu.VMEM((B_PER_W,), jnp.int32),
            pltpu.VMEM((N_CHUNKS, C_ROWS, D_EMBED), jnp.float32),
            pltpu.VMEM((B_PER_W * D_EMBED,), jnp.float32),
            pltpu.SemaphoreType.DMA,
            pltpu.SemaphoreType.DMA,
            pltpu.SemaphoreType.DMA,
            pltpu.SemaphoreType.DMA,
            pltpu.SemaphoreType.DMA,
            pltpu.SemaphoreType.DMA,
        ],
    )
    def emb_kernel(table_hbm, idx_hbm, pe_hbm, out_hbm,
                   idx_v, rows_v, pe_v, g0, g1, g2, g3, pe_sem, w_sem):
        wid = lax.axis_index("s") * NUM_CORES + lax.axis_index("c")
        b = wid // W_PER_BATCH
        col0 = lax.rem(wid, W_PER_BATCH) * B_PER_W
        pltpu.sync_copy(idx_hbm.at[b, pl.ds(col0, B_PER_W)], idx_v)
        gsems = [g0, g1, g2, g3]
        gathers = [
            pltpu.async_copy(
                table_hbm.at[idx_v.at[pl.ds(c * C_ROWS, C_ROWS)]],
                rows_v.at[c],
                gsems[c],
            )
            for c in range(N_CHUNKS)
        ]
        pe_off = pl.multiple_of(col0 * D_EMBED, 8)
        pe_cp = pltpu.async_copy(
            pe_hbm.at[pl.ds(pe_off, B_PER_W * D_EMBED)],
            pe_v, pe_sem)
        pe_cp.wait()
        writes = []
        for c in range(N_CHUNKS):
            gathers[c].wait()

            def row_body(p, carry, c=c):
                r0 = p * 2
                pe_base = (c * C_ROWS + r0) * D_EMBED
                for rr in range(2):
                    r = r0 + rr
                    for j in range(D_EMBED // LANES):
                        sl = pl.ds(j * LANES, LANES)
                        off = pe_base + rr * D_EMBED + j * LANES
                        rows_v[c, r, sl] = (rows_v[c, r, sl] * SCALE
                                            + pe_v[pl.ds(off, LANES)])
                return carry

            lax.fori_loop(0, C_ROWS // 2, row_body, 0)
            writes.append(pltpu.async_copy(
                rows_v.at[c],
                out_hbm.at[b, pl.ds(col0 + c * C_ROWS, C_ROWS)],
                w_sem,
            ))
        for w in writes:
            w.wait()

    return emb_kernel


_EMB_KERNEL = _make_sc_kernel()


def kernel(input, token_table):
    pe = jnp.asarray(_PE_BF16)
    return _EMB_KERNEL(token_table, input, pe)


# speedup vs baseline: 1.1087x; 1.0105x over previous
import functools
import math

import jax
import jax.numpy as jnp
import numpy as np
from jax import lax
from jax.experimental import pallas as pl
from jax.experimental.pallas import tpu as pltpu
from jax.experimental.pallas import tpu_sc as plsc

VOCAB = 1000000
SEQ_LEN = 2048
D_EMBED = 128
BATCH = 4
SCALE = math.sqrt(float(D_EMBED))

NUM_CORES = 2
NUM_SUBCORES = 16
NW = NUM_CORES * NUM_SUBCORES
B_TOTAL = BATCH * SEQ_LEN
B_PER_W = B_TOTAL // NW
W_PER_BATCH = SEQ_LEN // B_PER_W
N_CHUNKS = 4
C_ROWS = B_PER_W // N_CHUNKS
LANES = 16


def _positional_table() -> np.ndarray:
    pos = np.arange(SEQ_LEN)[:, None].astype(np.float32)
    i = np.arange(D_EMBED)[None, :].astype(np.float32)
    angle_rates = 1.0 / np.power(
        10000.0, (2.0 * np.floor(i / 2.0)) / float(D_EMBED))
    angles = pos * angle_rates
    pe = np.zeros((SEQ_LEN, D_EMBED), dtype=np.float32)
    pe[:, 0::2] = np.sin(angles[:, 0::2])
    pe[:, 1::2] = np.cos(angles[:, 1::2])
    return pe.reshape(SEQ_LEN * D_EMBED)


_PE_BF16 = _positional_table()


def _make_sc_kernel():
    mesh = plsc.VectorSubcoreMesh(
        core_axis_name="c", subcore_axis_name="s")

    @functools.partial(
        pl.kernel,
        mesh=mesh,
        out_type=jax.ShapeDtypeStruct((BATCH, SEQ_LEN, D_EMBED), jnp.float32),
        scratch_types=[
            pltpu.VMEM((B_PER_W,), jnp.int32),
            pltpu.VMEM((N_CHUNKS, C_ROWS, D_EMBED), jnp.float32),
            pltpu.VMEM((N_CHUNKS, C_ROWS, D_EMBED), jnp.float32),
            pltpu.VMEM((B_PER_W * D_EMBED,), jnp.float32),
            pltpu.SemaphoreType.DMA,
            pltpu.SemaphoreType.DMA,
            pltpu.SemaphoreType.DMA,
            pltpu.SemaphoreType.DMA,
            pltpu.SemaphoreType.DMA,
            pltpu.SemaphoreType.DMA,
        ],
    )
    def emb_kernel(table_hbm, idx_hbm, pe_hbm, out_hbm,
                   idx_v, rows_v, comp_v, pe_v, g0, g1, g2, g3,
                   pe_sem, w_sem):
        wid = lax.axis_index("s") * NUM_CORES + lax.axis_index("c")
        b = wid // W_PER_BATCH
        col0 = lax.rem(wid, W_PER_BATCH) * B_PER_W
        pltpu.sync_copy(idx_hbm.at[b, pl.ds(col0, B_PER_W)], idx_v)
        gsems = [g0, g1, g2, g3]
        gathers = [
            pltpu.async_copy(
                table_hbm.at[idx_v.at[pl.ds(c * C_ROWS, C_ROWS)]],
                rows_v.at[c],
                gsems[c],
            )
            for c in range(N_CHUNKS)
        ]
        pe_off = pl.multiple_of(col0 * D_EMBED, 8)
        pe_cp = pltpu.async_copy(
            pe_hbm.at[pl.ds(pe_off, B_PER_W * D_EMBED)],
            pe_v, pe_sem)
        pe_cp.wait()
        writes = []
        for c in range(N_CHUNKS):
            gathers[c].wait()

            def row_body(r, carry, c=c):
                pe_base = (c * C_ROWS + r) * D_EMBED
                for j in range(D_EMBED // LANES):
                    sl = pl.ds(j * LANES, LANES)
                    comp_v[c, r, sl] = (rows_v[c, r, sl] * SCALE
                                        + pe_v[pl.ds(pe_base + j * LANES,
                                                     LANES)])
                return carry

            lax.fori_loop(0, C_ROWS, row_body, 0)
            writes.append(pltpu.async_copy(
                comp_v.at[c],
                out_hbm.at[b, pl.ds(col0 + c * C_ROWS, C_ROWS)],
                w_sem,
            ))
        for w in writes:
            w.wait()

    return emb_kernel


_EMB_KERNEL = _make_sc_kernel()


def kernel(input, token_table):
    pe = jnp.asarray(_PE_BF16)
    return _EMB_KERNEL(token_table, input, pe)


# speedup vs baseline: 1.1420x; 1.0301x over previous
import functools
import math

import jax
import jax.numpy as jnp
import numpy as np
from jax import lax
from jax.experimental import pallas as pl
from jax.experimental.pallas import tpu as pltpu
from jax.experimental.pallas import tpu_sc as plsc

VOCAB = 1000000
SEQ_LEN = 2048
D_EMBED = 128
BATCH = 4
SCALE = math.sqrt(float(D_EMBED))

NUM_CORES = 2
NUM_SUBCORES = 16
NW = NUM_CORES * NUM_SUBCORES
B_TOTAL = BATCH * SEQ_LEN
B_PER_W = B_TOTAL // NW
W_PER_BATCH = SEQ_LEN // B_PER_W
N_CHUNKS = 4
C_ROWS = B_PER_W // N_CHUNKS
LANES = 16


def _positional_table() -> np.ndarray:
    pos = np.arange(SEQ_LEN)[:, None].astype(np.float32)
    i = np.arange(D_EMBED)[None, :].astype(np.float32)
    angle_rates = 1.0 / np.power(
        10000.0, (2.0 * np.floor(i / 2.0)) / float(D_EMBED))
    angles = pos * angle_rates
    pe = np.zeros((SEQ_LEN, D_EMBED), dtype=np.float32)
    pe[:, 0::2] = np.sin(angles[:, 0::2])
    pe[:, 1::2] = np.cos(angles[:, 1::2])
    return pe.reshape(SEQ_LEN * D_EMBED)


_PE_F32 = _positional_table()


def _make_sc_kernel():
    mesh = plsc.VectorSubcoreMesh(
        core_axis_name="c", subcore_axis_name="s")

    @functools.partial(
        pl.kernel,
        mesh=mesh,
        out_type=jax.ShapeDtypeStruct((BATCH, SEQ_LEN, D_EMBED), jnp.float32),
        scratch_types=[
            pltpu.VMEM((B_PER_W,), jnp.int32),
            pltpu.VMEM((N_CHUNKS, C_ROWS, D_EMBED), jnp.float32),
            pltpu.VMEM((N_CHUNKS, C_ROWS, D_EMBED), jnp.float32),
            pltpu.VMEM((B_PER_W * D_EMBED,), jnp.float32),
            pltpu.VMEM_SHARED((4, B_PER_W * D_EMBED), jnp.float32),
            pltpu.SemaphoreType.DMA,
            pltpu.SemaphoreType.DMA,
            pltpu.SemaphoreType.DMA,
            pltpu.SemaphoreType.DMA,
            pltpu.SemaphoreType.DMA,
        ],
    )
    def emb_kernel(table_hbm, idx_hbm, pe_hbm, out_hbm,
                   idx_v, rows_v, comp_v, pe_v, pe_spm, g0, g1, g2, g3,
                   w_sem):
        s = lax.axis_index("s")
        core = lax.axis_index("c")
        wid = s * NUM_CORES + core
        b = wid // W_PER_BATCH
        col0 = lax.rem(wid, W_PER_BATCH) * B_PER_W
        pltpu.sync_copy(idx_hbm.at[b, pl.ds(col0, B_PER_W)], idx_v)
        gsems = [g0, g1, g2, g3]
        gathers = [
            pltpu.async_copy(
                table_hbm.at[idx_v.at[pl.ds(c * C_ROWS, C_ROWS)]],
                rows_v.at[c],
                gsems[c],
            )
            for c in range(N_CHUNKS)
        ]
        @pl.when(s < 4)
        def _():
            pe_off = pl.multiple_of(col0 * D_EMBED, 8)
            pltpu.sync_copy(
                pe_hbm.at[pl.ds(pe_off, B_PER_W * D_EMBED)], pe_spm.at[s])

        plsc.subcore_barrier()
        pltpu.sync_copy(pe_spm.at[lax.rem(s, 4)], pe_v)
        writes = []
        for c in range(N_CHUNKS):
            gathers[c].wait()

            def row_body(r, carry, c=c):
                pe_base = (c * C_ROWS + r) * D_EMBED
                for j in range(D_EMBED // LANES):
                    sl = pl.ds(j * LANES, LANES)
                    comp_v[c, r, sl] = (rows_v[c, r, sl] * SCALE
                                        + pe_v[pl.ds(pe_base + j * LANES,
                                                     LANES)])
                return carry

            lax.fori_loop(0, C_ROWS, row_body, 0)
            writes.append(pltpu.async_copy(
                comp_v.at[c],
                out_hbm.at[b, pl.ds(col0 + c * C_ROWS, C_ROWS)],
                w_sem,
            ))
        for w in writes:
            w.wait()

    return emb_kernel


_EMB_KERNEL = _make_sc_kernel()


def kernel(input, token_table):
    pe = jnp.asarray(_PE_F32)
    return _EMB_KERNEL(token_table, input, pe)


# speedup vs baseline: 1.1639x; 1.0191x over previous
import functools
import math

import jax
import jax.numpy as jnp
import numpy as np
from jax import lax
from jax.experimental import pallas as pl
from jax.experimental.pallas import tpu as pltpu
from jax.experimental.pallas import tpu_sc as plsc

VOCAB = 1000000
SEQ_LEN = 2048
D_EMBED = 128
BATCH = 4
SCALE = math.sqrt(float(D_EMBED))

NUM_CORES = 2
NUM_SUBCORES = 16
NW = NUM_CORES * NUM_SUBCORES
B_TOTAL = BATCH * SEQ_LEN
B_PER_W = B_TOTAL // NW
W_PER_BATCH = SEQ_LEN // B_PER_W
N_CHUNKS = 4
C_ROWS = B_PER_W // N_CHUNKS
LANES = 16


def _positional_table() -> np.ndarray:
    pos = np.arange(SEQ_LEN)[:, None].astype(np.float32)
    i = np.arange(D_EMBED)[None, :].astype(np.float32)
    angle_rates = 1.0 / np.power(
        10000.0, (2.0 * np.floor(i / 2.0)) / float(D_EMBED))
    angles = pos * angle_rates
    pe = np.zeros((SEQ_LEN, D_EMBED), dtype=np.float32)
    pe[:, 0::2] = np.sin(angles[:, 0::2])
    pe[:, 1::2] = np.cos(angles[:, 1::2])
    return pe.reshape(SEQ_LEN * D_EMBED)


_PE_F32 = _positional_table()


def _make_sc_kernel():
    mesh = plsc.VectorSubcoreMesh(
        core_axis_name="c", subcore_axis_name="s")

    @functools.partial(
        pl.kernel,
        mesh=mesh,
        out_type=jax.ShapeDtypeStruct((BATCH, SEQ_LEN, D_EMBED), jnp.float32),
        scratch_types=[
            pltpu.VMEM((B_PER_W,), jnp.int32),
            pltpu.VMEM((N_CHUNKS, C_ROWS, D_EMBED), jnp.float32),
            pltpu.VMEM((N_CHUNKS, C_ROWS, D_EMBED), jnp.float32),
            pltpu.VMEM((B_PER_W * D_EMBED,), jnp.float32),
            pltpu.VMEM_SHARED((4, B_PER_W * D_EMBED), jnp.float32),
            pltpu.SemaphoreType.DMA,
            pltpu.SemaphoreType.DMA,
            pltpu.SemaphoreType.DMA,
            pltpu.SemaphoreType.DMA,
            pltpu.SemaphoreType.DMA,
            pltpu.SemaphoreType.DMA,
            pltpu.SemaphoreType.DMA,
        ],
    )
    def emb_kernel(table_hbm, idx_hbm, pe_hbm, out_hbm,
                   idx_v, rows_v, comp_v, pe_v, pe_spm, g0, g1, g2, g3,
                   w_sem, l_sem, pe_sem):
        s = lax.axis_index("s")
        core = lax.axis_index("c")
        wid = s * NUM_CORES + core
        b = wid // W_PER_BATCH
        col0 = lax.rem(wid, W_PER_BATCH) * B_PER_W
        pe_off = pl.multiple_of(col0 * D_EMBED, 8)

        @pl.when(s < 4)
        def _():
            pltpu.async_copy(
                pe_hbm.at[pl.ds(pe_off, B_PER_W * D_EMBED)], pe_spm.at[s],
                l_sem)

        pltpu.sync_copy(idx_hbm.at[b, pl.ds(col0, B_PER_W)], idx_v)
        gsems = [g0, g1, g2, g3]
        gathers = [
            pltpu.async_copy(
                table_hbm.at[idx_v.at[pl.ds(c * C_ROWS, C_ROWS)]],
                rows_v.at[c],
                gsems[c],
            )
            for c in range(N_CHUNKS)
        ]

        @pl.when(s < 4)
        def _():
            pltpu.make_async_copy(
                pe_hbm.at[pl.ds(pe_off, B_PER_W * D_EMBED)], pe_spm.at[s],
                l_sem).wait()

        plsc.subcore_barrier()
        pe_cp = pltpu.async_copy(pe_spm.at[lax.rem(s, 4)], pe_v, pe_sem)
        writes = []
        for c in range(N_CHUNKS):
            gathers[c].wait()
            if c == 0:
                pe_cp.wait()

            def row_body(r, carry, c=c):
                pe_base = (c * C_ROWS + r) * D_EMBED
                for j in range(D_EMBED // LANES):
                    sl = pl.ds(j * LANES, LANES)
                    comp_v[c, r, sl] = (rows_v[c, r, sl] * SCALE
                                        + pe_v[pl.ds(pe_base + j * LANES,
                                                     LANES)])
                return carry

            lax.fori_loop(0, C_ROWS, row_body, 0)
            writes.append(pltpu.async_copy(
                comp_v.at[c],
                out_hbm.at[b, pl.ds(col0 + c * C_ROWS, C_ROWS)],
                w_sem,
            ))
        for w in writes:
            w.wait()

    return emb_kernel


_EMB_KERNEL = _make_sc_kernel()


def kernel(input, token_table):
    pe = jnp.asarray(_PE_F32)
    return _EMB_KERNEL(token_table, input, pe)


# speedup vs baseline: 1.1691x; 1.0045x over previous
import functools
import math

import jax
import jax.numpy as jnp
import numpy as np
from jax import lax
from jax.experimental import pallas as pl
from jax.experimental.pallas import tpu as pltpu
from jax.experimental.pallas import tpu_sc as plsc

VOCAB = 1000000
SEQ_LEN = 2048
D_EMBED = 128
BATCH = 4
SCALE = math.sqrt(float(D_EMBED))

NUM_CORES = 2
NUM_SUBCORES = 16
NW = NUM_CORES * NUM_SUBCORES
B_TOTAL = BATCH * SEQ_LEN
B_PER_W = B_TOTAL // NW
W_PER_BATCH = SEQ_LEN // B_PER_W
N_CHUNKS = 4
C_ROWS = B_PER_W // N_CHUNKS
LANES = 16


_A_ROWS = 16
_B_ROWS = SEQ_LEN // _A_ROWS


def _pe_tc_body(out_ref):
    k = lax.broadcasted_iota(jnp.int32, (1, D_EMBED), 1)
    half = (k // 2).astype(jnp.float32)
    rate = jnp.exp(half * (-2.0 * math.log(10000.0) / float(D_EMBED)))
    phase = jnp.where(k % 2 == 0, 0.0, 0.5 * math.pi)

    a_ang = (lax.broadcasted_iota(jnp.int32, (_A_ROWS, D_EMBED), 0)
             .astype(jnp.float32) * float(_B_ROWS)) * rate + phase
    b_ang = (lax.broadcasted_iota(jnp.int32, (_B_ROWS, D_EMBED), 0)
             .astype(jnp.float32)) * rate
    sin_a, cos_a = jnp.sin(a_ang), jnp.cos(a_ang)
    sin_b, cos_b = jnp.sin(b_ang), jnp.cos(b_ang)
    pe = (sin_a[:, None, :] * cos_b[None, :, :]
          + cos_a[:, None, :] * sin_b[None, :, :])
    out_ref[...] = pe.reshape(SEQ_LEN * D_EMBED)


_pe_table_tc = pl.pallas_call(
    _pe_tc_body,
    out_shape=jax.ShapeDtypeStruct((SEQ_LEN * D_EMBED,), jnp.float32),
)


def _make_sc_kernel():
    mesh = plsc.VectorSubcoreMesh(
        core_axis_name="c", subcore_axis_name="s")

    @functools.partial(
        pl.kernel,
        mesh=mesh,
        out_type=jax.ShapeDtypeStruct((BATCH, SEQ_LEN, D_EMBED), jnp.float32),
        scratch_types=[
            pltpu.VMEM((B_PER_W,), jnp.int32),
            pltpu.VMEM((N_CHUNKS, C_ROWS, D_EMBED), jnp.float32),
            pltpu.VMEM((N_CHUNKS, C_ROWS, D_EMBED), jnp.float32),
            pltpu.VMEM((B_PER_W * D_EMBED,), jnp.float32),
            pltpu.VMEM_SHARED((4, B_PER_W * D_EMBED), jnp.float32),
            pltpu.SemaphoreType.DMA,
            pltpu.SemaphoreType.DMA,
            pltpu.SemaphoreType.DMA,
            pltpu.SemaphoreType.DMA,
            pltpu.SemaphoreType.DMA,
            pltpu.SemaphoreType.DMA,
            pltpu.SemaphoreType.DMA,
        ],
    )
    def emb_kernel(table_hbm, idx_hbm, pe_hbm, out_hbm,
                   idx_v, rows_v, comp_v, pe_v, pe_spm, g0, g1, g2, g3,
                   w_sem, l_sem, pe_sem):
        s = lax.axis_index("s")
        core = lax.axis_index("c")
        wid = s * NUM_CORES + core
        b = wid // W_PER_BATCH
        col0 = lax.rem(wid, W_PER_BATCH) * B_PER_W
        pe_off = pl.multiple_of(col0 * D_EMBED, 8)

        @pl.when(s < 4)
        def _():
            pltpu.async_copy(
                pe_hbm.at[pl.ds(pe_off, B_PER_W * D_EMBED)], pe_spm.at[s],
                l_sem)

        pltpu.sync_copy(idx_hbm.at[b, pl.ds(col0, B_PER_W)], idx_v)
        gsems = [g0, g1, g2, g3]
        gathers = [
            pltpu.async_copy(
                table_hbm.at[idx_v.at[pl.ds(c * C_ROWS, C_ROWS)]],
                rows_v.at[c],
                gsems[c],
            )
            for c in range(N_CHUNKS)
        ]

        @pl.when(s < 4)
        def _():
            pltpu.make_async_copy(
                pe_hbm.at[pl.ds(pe_off, B_PER_W * D_EMBED)], pe_spm.at[s],
                l_sem).wait()

        plsc.subcore_barrier()
        pe_cp = pltpu.async_copy(pe_spm.at[lax.rem(s, 4)], pe_v, pe_sem)
        writes = []
        for c in range(N_CHUNKS):
            gathers[c].wait()
            if c == 0:
                pe_cp.wait()

            def row_body(r, carry, c=c):
                pe_base = (c * C_ROWS + r) * D_EMBED
                for j in range(D_EMBED // LANES):
                    sl = pl.ds(j * LANES, LANES)
                    comp_v[c, r, sl] = (rows_v[c, r, sl] * SCALE
                                        + pe_v[pl.ds(pe_base + j * LANES,
                                                     LANES)])
                return carry

            lax.fori_loop(0, C_ROWS, row_body, 0)
            writes.append(pltpu.async_copy(
                comp_v.at[c],
                out_hbm.at[b, pl.ds(col0 + c * C_ROWS, C_ROWS)],
                w_sem,
            ))
        for w in writes:
            w.wait()

    return emb_kernel


_EMB_KERNEL = _make_sc_kernel()


def kernel(input, token_table):
    pe = _pe_table_tc()
    return _EMB_KERNEL(token_table, input, pe)


# speedup vs baseline: 1.1768x; 1.0066x over previous
import functools
import math

import jax
import jax.numpy as jnp
import numpy as np
from jax import lax
from jax.experimental import pallas as pl
from jax.experimental.pallas import tpu as pltpu
from jax.experimental.pallas import tpu_sc as plsc

VOCAB = 1000000
SEQ_LEN = 2048
D_EMBED = 128
BATCH = 4
SCALE = math.sqrt(float(D_EMBED))

NUM_CORES = 2
NUM_SUBCORES = 16
NW = NUM_CORES * NUM_SUBCORES
B_TOTAL = BATCH * SEQ_LEN
B_PER_W = B_TOTAL // NW
W_PER_BATCH = SEQ_LEN // B_PER_W
N_CHUNKS = 4
C_ROWS = B_PER_W // N_CHUNKS
LANES = 16


_A_ROWS = 16
_B_ROWS = SEQ_LEN // _A_ROWS


def _pe_tc_body(out_ref):
    k = lax.broadcasted_iota(jnp.int32, (1, D_EMBED), 1)
    half = (k // 2).astype(jnp.float32)
    rate = jnp.exp(half * (-2.0 * math.log(10000.0) / float(D_EMBED)))
    phase = jnp.where(k % 2 == 0, 0.0, 0.5 * math.pi)

    a_ang = (lax.broadcasted_iota(jnp.int32, (_A_ROWS, D_EMBED), 0)
             .astype(jnp.float32) * float(_B_ROWS)) * rate + phase
    b_ang = (lax.broadcasted_iota(jnp.int32, (_B_ROWS, D_EMBED), 0)
             .astype(jnp.float32)) * rate
    sin_a, cos_a = jnp.sin(a_ang), jnp.cos(a_ang)
    sin_b, cos_b = jnp.sin(b_ang), jnp.cos(b_ang)
    pe = (sin_a[:, None, :] * cos_b[None, :, :]
          + cos_a[:, None, :] * sin_b[None, :, :])
    out_ref[...] = pe.reshape(SEQ_LEN * D_EMBED)


_pe_table_tc = pl.pallas_call(
    _pe_tc_body,
    out_shape=jax.ShapeDtypeStruct((SEQ_LEN * D_EMBED,), jnp.float32),
)


def _make_sc_kernel():
    mesh = plsc.VectorSubcoreMesh(
        core_axis_name="c", subcore_axis_name="s")

    @functools.partial(
        pl.kernel,
        mesh=mesh,
        out_type=jax.ShapeDtypeStruct((BATCH, SEQ_LEN, D_EMBED), jnp.float32),
        scratch_types=[
            pltpu.VMEM((B_PER_W,), jnp.int32),
            pltpu.VMEM((N_CHUNKS, C_ROWS, D_EMBED), jnp.float32),
            pltpu.VMEM((N_CHUNKS, C_ROWS, D_EMBED), jnp.float32),
            pltpu.VMEM((B_PER_W * D_EMBED,), jnp.float32),
            pltpu.VMEM_SHARED((4, B_PER_W * D_EMBED), jnp.float32),
            pltpu.SemaphoreType.DMA,
            pltpu.SemaphoreType.DMA,
            pltpu.SemaphoreType.DMA,
            pltpu.SemaphoreType.DMA,
            pltpu.SemaphoreType.DMA,
            pltpu.SemaphoreType.DMA,
            pltpu.SemaphoreType.DMA,
            pltpu.SemaphoreType.DMA,
            pltpu.SemaphoreType.DMA,
            pltpu.SemaphoreType.DMA,
        ],
    )
    def emb_kernel(table_hbm, idx_hbm, pe_hbm, out_hbm,
                   idx_v, rows_v, comp_v, pe_v, pe_spm, g0, g1, g2, g3,
                   w_sem, l_sem, p0, p1, p2, p3):
        s = lax.axis_index("s")
        core = lax.axis_index("c")
        wid = s * NUM_CORES + core
        b = wid // W_PER_BATCH
        col0 = lax.rem(wid, W_PER_BATCH) * B_PER_W
        pe_off = pl.multiple_of(col0 * D_EMBED, 8)

        @pl.when(s < 4)
        def _():
            pltpu.async_copy(
                pe_hbm.at[pl.ds(pe_off, B_PER_W * D_EMBED)], pe_spm.at[s],
                l_sem)

        pltpu.sync_copy(idx_hbm.at[b, pl.ds(col0, B_PER_W)], idx_v)
        gsems = [g0, g1, g2, g3]
        gathers = [
            pltpu.async_copy(
                table_hbm.at[idx_v.at[pl.ds(c * C_ROWS, C_ROWS)]],
                rows_v.at[c],
                gsems[c],
            )
            for c in range(N_CHUNKS)
        ]

        @pl.when(s < 4)
        def _():
            pltpu.make_async_copy(
                pe_hbm.at[pl.ds(pe_off, B_PER_W * D_EMBED)], pe_spm.at[s],
                l_sem).wait()

        plsc.subcore_barrier()
        slot = lax.rem(s, 4)
        psems = [p0, p1, p2, p3]
        pe_cps = [
            pltpu.async_copy(
                pe_spm.at[slot, pl.ds(c * C_ROWS * D_EMBED,
                                      C_ROWS * D_EMBED)],
                pe_v.at[pl.ds(c * C_ROWS * D_EMBED, C_ROWS * D_EMBED)],
                psems[c],
            )
            for c in range(N_CHUNKS)
        ]
        writes = []
        for c in range(N_CHUNKS):
            gathers[c].wait()
            pe_cps[c].wait()

            def row_body(r, carry, c=c):
                pe_base = (c * C_ROWS + r) * D_EMBED
                for j in range(D_EMBED // LANES):
                    sl = pl.ds(j * LANES, LANES)
                    comp_v[c, r, sl] = (rows_v[c, r, sl] * SCALE
                                        + pe_v[pl.ds(pe_base + j * LANES,
                                                     LANES)])
                return carry

            lax.fori_loop(0, C_ROWS, row_body, 0)
            writes.append(pltpu.async_copy(
                comp_v.at[c],
                out_hbm.at[b, pl.ds(col0 + c * C_ROWS, C_ROWS)],
                w_sem,
            ))
        for w in writes:
            w.wait()

    return emb_kernel


_EMB_KERNEL = _make_sc_kernel()


def kernel(input, token_table):
    pe = _pe_table_tc()
    return _EMB_KERNEL(token_table, input, pe)
